# baseline (speedup 1.0000x reference)
import functools

import jax
import jax.numpy as jnp
from jax import lax
from jax.experimental import pallas as pl
from jax.experimental.pallas import tpu as pltpu
from jax.experimental.pallas import tpu_sc as plsc

N_SEG, P, D, K, N_LABELS = 64, 512, 64, 15, 16
LANES = 16
NCHUNK = P // LANES


def _sc_body(center_hbm, outputs_hbm, labels_hbm, out_hbm,
             pts0_v, pts1_v, cen_v, lab_v, dd_v, cnt_v, res_v,
             sem0, sem1):
    nc = 2
    wid = lax.axis_index("s") * nc + lax.axis_index("c")
    iota = lax.iota(jnp.int32, LANES)

    sum_dd_total = jnp.float32(0.0)
    push_total = jnp.int32(0)

    seg0 = wid * 2
    cp0 = pltpu.async_copy(outputs_hbm.at[seg0], pts0_v, sem0)
    cp1 = pltpu.async_copy(outputs_hbm.at[seg0 + 1], pts1_v, sem1)

    for s, (pts_v, cp) in enumerate(((pts0_v, cp0), (pts1_v, cp1))):
        seg = seg0 + s
        pltpu.sync_copy(center_hbm.at[seg], cen_v)
        pltpu.sync_copy(labels_hbm.at[seg], lab_v)
        cp.wait()

        c_regs = [cen_v[pl.ds(16 * q, 16)] for q in range(4)]

        def chunk_body(g, carry):
            cnt_tab, macc, sacc = carry
            base = g * LANES
            acc = jnp.zeros((LANES,), jnp.float32)
            for d in range(D):
                t = pts_v[d, pl.ds(base, 16)] - c_regs[d // 16][d % 16]
                acc = acc + t * t
            dd_v[pl.ds(base, LANES)] = acc
            lv = lab_v[pl.ds(base, LANES)]
            for j in range(LANES):
                onehot = iota == lv[j]
                cnt_tab = cnt_tab + onehot.astype(jnp.int32)
                take = jnp.logical_and(onehot, cnt_tab <= K)
                macc = jnp.where(take, jnp.maximum(macc, acc[j]), macc)
            return cnt_tab, macc, sacc + acc

        cnt_tab, macc, sacc = lax.fori_loop(
            0, NCHUNK, chunk_body,
            (jnp.zeros((LANES,), jnp.int32),
             jnp.full((LANES,), -jnp.inf, jnp.float32),
             jnp.zeros((LANES,), jnp.float32)))
        sum_dd_total = sum_dd_total + jnp.sum(sacc)

        lv0 = lab_v[pl.ds(0, LANES)]
        dd0 = dd_v[pl.ds(0, LANES)]
        need = K - cnt_tab
        active = jnp.logical_and(cnt_tab > 0, need > 0)
        t_vec = jnp.zeros((LANES,), jnp.int32)
        for j in range(K):
            notl = iota != lv0[j]
            t_vec = t_vec + notl.astype(jnp.int32)
            take = jnp.logical_and(active,
                                   jnp.logical_and(notl, t_vec <= need))
            macc = jnp.where(take, jnp.maximum(macc, dd0[j]), macc)

        margin = jnp.float32(1.0) + jnp.max(macc)
        cnt_v[...] = cnt_tab

        def push_group(g, pacc):
            ddc = dd_v[pl.ds(g * LANES, LANES)]
            lv = lab_v[pl.ds(g * LANES, LANES)]
            cv = plsc.load_gather(cnt_v, [lv])
            w = jnp.where(ddc < margin, jnp.int32(P) - cv,
                          jnp.zeros((LANES,), jnp.int32))
            return pacc + w

        pacc = lax.fori_loop(0, NCHUNK, push_group,
                             jnp.zeros((LANES,), jnp.int32))
        push_total = push_total + jnp.sum(pacc)

    res = jnp.where(iota == 0, sum_dd_total,
                    jnp.where(iota == 1, push_total.astype(jnp.float32),
                              jnp.float32(0.0)))
    res_v[...] = res
    pltpu.sync_copy(res_v, out_hbm.at[wid])


@jax.jit
def _lmnn_sc(segment_center, outputs, label_inds):
    mesh = plsc.VectorSubcoreMesh(core_axis_name="c", subcore_axis_name="s")
    f = functools.partial(
        pl.kernel,
        out_type=jax.ShapeDtypeStruct((32, LANES), jnp.float32),
        mesh=mesh,
        compiler_params=pltpu.CompilerParams(needs_layout_passes=False),
        scratch_types=[
            pltpu.VMEM((D, P), jnp.float32),
            pltpu.VMEM((D, P), jnp.float32),
            pltpu.VMEM((D,), jnp.float32),
            pltpu.VMEM((P,), jnp.int32),
            pltpu.VMEM((P,), jnp.float32),
            pltpu.VMEM((N_LABELS,), jnp.int32),
            pltpu.VMEM((LANES,), jnp.float32),
            pltpu.SemaphoreType.DMA,
            pltpu.SemaphoreType.DMA,
        ],
    )(_sc_body)
    out = f(segment_center, outputs.transpose(0, 2, 1), label_inds)
    pull = jnp.float32(K) * jnp.sum(out[:, 0])
    push = jnp.sum(out[:, 1])
    return (pull + push) / jnp.float32(N_SEG * P)


def kernel(segment_center, outputs, label_inds):
    return _lmnn_sc(segment_center, outputs, label_inds)


# speedup vs baseline: 1.0175x; 1.0175x over previous
import functools

import jax
import jax.numpy as jnp
from jax import lax
from jax.experimental import pallas as pl
from jax.experimental.pallas import tpu as pltpu
from jax.experimental.pallas import tpu_sc as plsc

N_SEG, P, D, K, N_LABELS = 64, 512, 64, 15, 16
LANES = 16
NCHUNK = P // LANES


def _sc_body(center_hbm, outputs_hbm, labels_hbm, out_hbm,
             pts0_v, pts1_v, cen_v, lab_v, dd_v, cnt_v, res_v,
             sem0, sem1):
    nc = 2
    wid = lax.axis_index("s") * nc + lax.axis_index("c")
    iota = lax.iota(jnp.int32, LANES)

    sum_dd_total = jnp.float32(0.0)
    push_total = jnp.int32(0)

    seg0 = wid * 2
    cp0 = pltpu.async_copy(outputs_hbm.at[seg0], pts0_v, sem0)
    cp1 = pltpu.async_copy(outputs_hbm.at[seg0 + 1], pts1_v, sem1)

    for s, (pts_v, cp) in enumerate(((pts0_v, cp0), (pts1_v, cp1))):
        seg = seg0 + s
        pltpu.sync_copy(center_hbm.at[seg], cen_v)
        pltpu.sync_copy(labels_hbm.at[seg], lab_v)
        cp.wait()

        c_regs = [cen_v[pl.ds(16 * q, 16)] for q in range(4)]

        def one_chunk(base, cnt_tab, macc, sacc):
            acc = jnp.zeros((LANES,), jnp.float32)
            for d in range(D):
                t = pts_v[d, pl.ds(base, 16)] - c_regs[d // 16][d % 16]
                acc = acc + t * t
            dd_v[pl.ds(base, LANES)] = acc
            lv = lab_v[pl.ds(base, LANES)]
            for j in range(LANES):
                onehot = iota == lv[j]
                cnt_tab = cnt_tab + onehot.astype(jnp.int32)
                take = jnp.logical_and(onehot, cnt_tab <= K)
                macc = jnp.where(take, jnp.maximum(macc, acc[j]), macc)
            return cnt_tab, macc, sacc + acc

        def chunk_body(g, carry):
            cnt_tab, macc, sacc = carry
            base = g * (2 * LANES)
            cnt_tab, macc, sacc = one_chunk(base, cnt_tab, macc, sacc)
            cnt_tab, macc, sacc = one_chunk(base + LANES, cnt_tab, macc, sacc)
            return cnt_tab, macc, sacc

        cnt_tab, macc, sacc = lax.fori_loop(
            0, NCHUNK // 2, chunk_body,
            (jnp.zeros((LANES,), jnp.int32),
             jnp.full((LANES,), -jnp.inf, jnp.float32),
             jnp.zeros((LANES,), jnp.float32)))
        sum_dd_total = sum_dd_total + jnp.sum(sacc)

        lv0 = lab_v[pl.ds(0, LANES)]
        dd0 = dd_v[pl.ds(0, LANES)]
        need = K - cnt_tab
        active = jnp.logical_and(cnt_tab > 0, need > 0)
        t_vec = jnp.zeros((LANES,), jnp.int32)
        for j in range(K):
            notl = iota != lv0[j]
            t_vec = t_vec + notl.astype(jnp.int32)
            take = jnp.logical_and(active,
                                   jnp.logical_and(notl, t_vec <= need))
            macc = jnp.where(take, jnp.maximum(macc, dd0[j]), macc)

        margin = jnp.float32(1.0) + jnp.max(macc)
        cnt_v[...] = cnt_tab

        def push_group(g, pacc):
            for u in range(4):
                off = g * (4 * LANES) + u * LANES
                ddc = dd_v[pl.ds(off, LANES)]
                lv = lab_v[pl.ds(off, LANES)]
                cv = plsc.load_gather(cnt_v, [lv])
                w = jnp.where(ddc < margin, jnp.int32(P) - cv,
                              jnp.zeros((LANES,), jnp.int32))
                pacc = pacc + w
            return pacc

        pacc = lax.fori_loop(0, NCHUNK // 4, push_group,
                             jnp.zeros((LANES,), jnp.int32))
        push_total = push_total + jnp.sum(pacc)

    res = jnp.where(iota == 0, sum_dd_total,
                    jnp.where(iota == 1, push_total.astype(jnp.float32),
                              jnp.float32(0.0)))
    res_v[...] = res
    pltpu.sync_copy(res_v, out_hbm.at[wid])


@jax.jit
def _lmnn_sc(segment_center, outputs, label_inds):
    mesh = plsc.VectorSubcoreMesh(core_axis_name="c", subcore_axis_name="s")
    f = functools.partial(
        pl.kernel,
        out_type=jax.ShapeDtypeStruct((32, LANES), jnp.float32),
        mesh=mesh,
        compiler_params=pltpu.CompilerParams(needs_layout_passes=False),
        scratch_types=[
            pltpu.VMEM((D, P), jnp.float32),
            pltpu.VMEM((D, P), jnp.float32),
            pltpu.VMEM((D,), jnp.float32),
            pltpu.VMEM((P,), jnp.int32),
            pltpu.VMEM((P,), jnp.float32),
            pltpu.VMEM((N_LABELS,), jnp.int32),
            pltpu.VMEM((LANES,), jnp.float32),
            pltpu.SemaphoreType.DMA,
            pltpu.SemaphoreType.DMA,
        ],
    )(_sc_body)
    out = f(segment_center, outputs.transpose(0, 2, 1), label_inds)
    pull = jnp.float32(K) * jnp.sum(out[:, 0])
    push = jnp.sum(out[:, 1])
    return (pull + push) / jnp.float32(N_SEG * P)


def kernel(segment_center, outputs, label_inds):
    return _lmnn_sc(segment_center, outputs, label_inds)


# speedup vs baseline: 1.0426x; 1.0247x over previous
import functools

import jax
import jax.numpy as jnp
from jax import lax
from jax.experimental import pallas as pl
from jax.experimental.pallas import tpu as pltpu
from jax.experimental.pallas import tpu_sc as plsc

N_SEG, P, D, K, N_LABELS = 64, 512, 64, 15, 16
LANES = 16
NCHUNK = P // LANES


def _sc_body(center_hbm, outputs_hbm, labels_hbm, out_hbm,
             pts0_v, pts1_v, cen0_v, cen1_v, lab0_v, lab1_v,
             dd_v, cnt_v, res_v,
             sem0, sem1, sem2, sem3, sem4, sem5):
    nc = 2
    wid = lax.axis_index("s") * nc + lax.axis_index("c")
    iota = lax.iota(jnp.int32, LANES)

    sum_dd_total = jnp.float32(0.0)
    push_total = jnp.int32(0)

    seg0 = wid * 2
    cp0 = pltpu.async_copy(outputs_hbm.at[seg0], pts0_v, sem0)
    cp1 = pltpu.async_copy(outputs_hbm.at[seg0 + 1], pts1_v, sem1)
    cpc0 = pltpu.async_copy(center_hbm.at[seg0], cen0_v, sem2)
    cpc1 = pltpu.async_copy(center_hbm.at[seg0 + 1], cen1_v, sem3)
    cpl0 = pltpu.async_copy(labels_hbm.at[seg0], lab0_v, sem4)
    cpl1 = pltpu.async_copy(labels_hbm.at[seg0 + 1], lab1_v, sem5)

    for s, (pts_v, cen_v, lab_v, cps) in enumerate((
            (pts0_v, cen0_v, lab0_v, (cp0, cpc0, cpl0)),
            (pts1_v, cen1_v, lab1_v, (cp1, cpc1, cpl1)))):
        for cp in cps:
            cp.wait()

        c_regs = [cen_v[pl.ds(16 * q, 16)] for q in range(4)]

        def one_chunk(base, cnt_tab, macc, sacc):
            acc = jnp.zeros((LANES,), jnp.float32)
            for d in range(D):
                t = pts_v[d, pl.ds(base, 16)] - c_regs[d // 16][d % 16]
                acc = acc + t * t
            dd_v[pl.ds(base, LANES)] = acc
            lv = lab_v[pl.ds(base, LANES)]
            for j in range(LANES):
                onehot = iota == lv[j]
                cnt_tab = cnt_tab + onehot.astype(jnp.int32)
                take = jnp.logical_and(onehot, cnt_tab <= K)
                macc = jnp.where(take, jnp.maximum(macc, acc[j]), macc)
            return cnt_tab, macc, sacc + acc

        def chunk_body(g, carry):
            cnt_tab, macc, sacc = carry
            base = g * (2 * LANES)
            cnt_tab, macc, sacc = one_chunk(base, cnt_tab, macc, sacc)
            cnt_tab, macc, sacc = one_chunk(base + LANES, cnt_tab, macc, sacc)
            return cnt_tab, macc, sacc

        cnt_tab, macc, sacc = lax.fori_loop(
            0, NCHUNK // 2, chunk_body,
            (jnp.zeros((LANES,), jnp.int32),
             jnp.full((LANES,), -jnp.inf, jnp.float32),
             jnp.zeros((LANES,), jnp.float32)))
        sum_dd_total = sum_dd_total + jnp.sum(sacc)

        lv0 = lab_v[pl.ds(0, LANES)]
        dd0 = dd_v[pl.ds(0, LANES)]
        need = K - cnt_tab
        active = jnp.logical_and(cnt_tab > 0, need > 0)
        t_vec = jnp.zeros((LANES,), jnp.int32)
        for j in range(K):
            notl = iota != lv0[j]
            t_vec = t_vec + notl.astype(jnp.int32)
            take = jnp.logical_and(active,
                                   jnp.logical_and(notl, t_vec <= need))
            macc = jnp.where(take, jnp.maximum(macc, dd0[j]), macc)

        margin = jnp.float32(1.0) + jnp.max(macc)
        cnt_v[...] = cnt_tab

        def push_group(g, pacc):
            for u in range(4):
                off = g * (4 * LANES) + u * LANES
                ddc = dd_v[pl.ds(off, LANES)]
                lv = lab_v[pl.ds(off, LANES)]
                cv = plsc.load_gather(cnt_v, [lv])
                w = jnp.where(ddc < margin, jnp.int32(P) - cv,
                              jnp.zeros((LANES,), jnp.int32))
                pacc = pacc + w
            return pacc

        pacc = lax.fori_loop(0, NCHUNK // 4, push_group,
                             jnp.zeros((LANES,), jnp.int32))
        push_total = push_total + jnp.sum(pacc)

    res = jnp.where(iota == 0, sum_dd_total,
                    jnp.where(iota == 1, push_total.astype(jnp.float32),
                              jnp.float32(0.0)))
    res_v[...] = res
    pltpu.sync_copy(res_v, out_hbm.at[wid])


@jax.jit
def _lmnn_sc(segment_center, outputs, label_inds):
    mesh = plsc.VectorSubcoreMesh(core_axis_name="c", subcore_axis_name="s")
    f = functools.partial(
        pl.kernel,
        out_type=jax.ShapeDtypeStruct((32, LANES), jnp.float32),
        mesh=mesh,
        compiler_params=pltpu.CompilerParams(needs_layout_passes=False),
        scratch_types=[
            pltpu.VMEM((D, P), jnp.float32),
            pltpu.VMEM((D, P), jnp.float32),
            pltpu.VMEM((D,), jnp.float32),
            pltpu.VMEM((D,), jnp.float32),
            pltpu.VMEM((P,), jnp.int32),
            pltpu.VMEM((P,), jnp.int32),
            pltpu.VMEM((P,), jnp.float32),
            pltpu.VMEM((N_LABELS,), jnp.int32),
            pltpu.VMEM((LANES,), jnp.float32),
        ] + [pltpu.SemaphoreType.DMA] * 6,
    )(_sc_body)
    out = f(segment_center, outputs.transpose(0, 2, 1), label_inds)
    pull = jnp.float32(K) * jnp.sum(out[:, 0])
    push = jnp.sum(out[:, 1])
    return (pull + push) / jnp.float32(N_SEG * P)


def kernel(segment_center, outputs, label_inds):
    return _lmnn_sc(segment_center, outputs, label_inds)
